# baseline (speedup 1.0000x reference)
import functools

import jax
import jax.numpy as jnp
from jax import lax
from jax.experimental import pallas as pl
from jax.experimental.pallas import tpu as pltpu
from jax.experimental.pallas import tpu_sc as plsc

_TAU = 0.1
_EPS = 1e-8
_LANES = 16


def _topk_body(x_ref, w1_ref, b1_ref, w2_ref, b2_ref, g_ref, idx_ref, w_ref):
    x = x_ref[...]
    h = jnp.maximum(
        lax.dot_general(x, w1_ref[...], (((1,), (0,)), ((), ())),
                        preferred_element_type=jnp.float32)
        + b1_ref[...][None, :], 0.0)
    q = lax.dot_general(h, w2_ref[...], (((1,), (0,)), ((), ())),
                        preferred_element_type=jnp.float32) + b2_ref[...][None, :]
    qn = q / jnp.maximum(jnp.sqrt(jnp.sum(q * q, axis=1, keepdims=True)), _EPS)
    g = g_ref[...]
    gn = g / jnp.maximum(jnp.sqrt(jnp.sum(g * g, axis=1, keepdims=True)), _EPS)
    sim = lax.dot_general(qn, gn, (((1,), (1,)), ((), ())),
                          preferred_element_type=jnp.float32)
    n = sim.shape[1]
    ii = lax.broadcasted_iota(jnp.int32, sim.shape, 1)
    m1 = jnp.max(sim, axis=1, keepdims=True)
    i1 = jnp.min(jnp.where(sim == m1, ii, n), axis=1, keepdims=True)
    sim2 = jnp.where(ii == i1, -jnp.inf, sim)
    m2 = jnp.max(sim2, axis=1, keepdims=True)
    i2 = jnp.min(jnp.where(sim2 == m2, ii, n), axis=1, keepdims=True)
    e = jnp.exp((m2 - m1) / _TAU)
    denom = 1.0 + e
    idx_ref[...] = jnp.concatenate([i1, i2], axis=1)
    w_ref[...] = jnp.concatenate([1.0 / denom, e / denom], axis=1)


def _topk_tc(query_encoding, W1, b1, W2, b2, gallery_embeddings):
    B = query_encoding.shape[0]
    return pl.pallas_call(
        _topk_body,
        out_shape=(
            jax.ShapeDtypeStruct((B, 2), jnp.int32),
            jax.ShapeDtypeStruct((B, 2), jnp.float32),
        ),
    )(query_encoding, W1, b1, W2, b2, gallery_embeddings)


def _gather_combine_sc(imgs2, masks2, idxp, wb):
    nrow = idxp.shape[0]
    dh = imgs2.shape[1]
    mh = masks2.shape[1]
    mesh = plsc.VectorSubcoreMesh(core_axis_name="c", subcore_axis_name="s")

    @functools.partial(
        pl.kernel,
        out_type=(
            jax.ShapeDtypeStruct((nrow, dh), jnp.float32),
            jax.ShapeDtypeStruct((nrow, mh), jnp.float32),
        ),
        mesh=mesh,
        scratch_types=[
            pltpu.VMEM((8,), jnp.int32),
            pltpu.VMEM((2, _LANES), jnp.float32),
            pltpu.VMEM((2, dh), jnp.float32),
            pltpu.VMEM((2, mh), jnp.float32),
            pltpu.VMEM((dh,), jnp.float32),
            pltpu.VMEM((mh,), jnp.float32),
            pltpu.SemaphoreType.DMA,
            pltpu.SemaphoreType.DMA,
        ],
    )
    def k(imgs_hbm, masks_hbm, idx_hbm, wb_hbm, gi_hbm, gm_hbm,
          idx_v, w_v, img_rows, msk_rows, img_acc, msk_acc, sem1, sem2):
        wid = lax.axis_index("s") * 2 + lax.axis_index("c")
        pltpu.sync_copy(idx_hbm.at[wid], idx_v)
        pltpu.sync_copy(wb_hbm.at[wid], w_v)
        cp1 = pltpu.async_copy(imgs_hbm.at[idx_v.at[pl.ds(0, 2)]], img_rows, sem1)
        cp2 = pltpu.async_copy(masks_hbm.at[idx_v.at[pl.ds(0, 2)]], msk_rows, sem2)
        w0 = w_v[0]
        w1 = w_v[1]
        cp1.wait()

        def body_img(i, carry):
            s = pl.ds(i * _LANES, _LANES)
            img_acc[s] = w0 * img_rows[0, s] + w1 * img_rows[1, s]
            return carry

        lax.fori_loop(0, dh // _LANES, body_img, 0)
        cp2.wait()

        def body_msk(i, carry):
            s = pl.ds(i * _LANES, _LANES)
            msk_acc[s] = w0 * msk_rows[0, s] + w1 * msk_rows[1, s]
            return carry

        lax.fori_loop(0, mh // _LANES, body_msk, 0)
        pltpu.sync_copy(img_acc, gi_hbm.at[wid])
        pltpu.sync_copy(msk_acc, gm_hbm.at[wid])

    return k(imgs2, masks2, idxp, wb)


def kernel(query_encoding, W1, b1, W2, b2, gallery_embeddings,
           gallery_images, gallery_masks):
    B = query_encoding.shape[0]
    N, C, H, W = gallery_images.shape
    D = C * H * W
    M = H * W

    topk_idx, topk_w = _topk_tc(query_encoding, W1, b1, W2, b2,
                                gallery_embeddings)

    rep_idx = jnp.repeat(topk_idx, 2, axis=0)
    half = (jnp.arange(2 * B, dtype=jnp.int32) % 2)[:, None]
    idxp = jnp.concatenate(
        [2 * rep_idx + half,
         jnp.zeros((2 * B, 6), dtype=jnp.int32)], axis=1)
    wb = jnp.broadcast_to(
        jnp.repeat(topk_w, 2, axis=0)[:, :, None], (2 * B, 2, _LANES))

    imgs2 = gallery_images.reshape(2 * N, D // 2)
    masks2 = gallery_masks.reshape(2 * N, M // 2)
    gi2, gm2 = _gather_combine_sc(imgs2, masks2, idxp, wb)

    guide_image = gi2.reshape(B, C, H, W)
    guide_mask = gm2.reshape(B, H, W)
    return guide_image, guide_mask


# speedup vs baseline: 1.0187x; 1.0187x over previous
import functools

import jax
import jax.numpy as jnp
from jax import lax
from jax.experimental import pallas as pl
from jax.experimental.pallas import tpu as pltpu
from jax.experimental.pallas import tpu_sc as plsc

_TAU = 0.1
_EPS = 1e-8


def _topk_body(x_ref, w1_ref, b1_ref, w2_ref, b2_ref, g_ref, idx_ref, w_ref):
    x = x_ref[...]
    h = jnp.maximum(
        lax.dot_general(x, w1_ref[...], (((1,), (0,)), ((), ())),
                        preferred_element_type=jnp.float32)
        + b1_ref[...][None, :], 0.0)
    q = lax.dot_general(h, w2_ref[...], (((1,), (0,)), ((), ())),
                        preferred_element_type=jnp.float32) + b2_ref[...][None, :]
    qn = q / jnp.maximum(jnp.sqrt(jnp.sum(q * q, axis=1, keepdims=True)), _EPS)
    g = g_ref[...]
    gn = g / jnp.maximum(jnp.sqrt(jnp.sum(g * g, axis=1, keepdims=True)), _EPS)
    sim = lax.dot_general(qn, gn, (((1,), (1,)), ((), ())),
                          preferred_element_type=jnp.float32)
    n = sim.shape[1]
    ii = lax.broadcasted_iota(jnp.int32, sim.shape, 1)
    m1 = jnp.max(sim, axis=1, keepdims=True)
    i1 = jnp.min(jnp.where(sim == m1, ii, n), axis=1, keepdims=True)
    sim2 = jnp.where(ii == i1, -jnp.inf, sim)
    m2 = jnp.max(sim2, axis=1, keepdims=True)
    i2 = jnp.min(jnp.where(sim2 == m2, ii, n), axis=1, keepdims=True)
    e = jnp.exp((m2 - m1) / _TAU)
    denom = 1.0 + e
    idx_ref[...] = jnp.concatenate([i1, i2], axis=1)
    w_ref[...] = jnp.concatenate([1.0 / denom, e / denom], axis=1)


def _topk_tc(query_encoding, W1, b1, W2, b2, gallery_embeddings):
    B = query_encoding.shape[0]
    return pl.pallas_call(
        _topk_body,
        out_shape=(
            jax.ShapeDtypeStruct((B, 2), jnp.int32),
            jax.ShapeDtypeStruct((B, 2), jnp.float32),
        ),
    )(query_encoding, W1, b1, W2, b2, gallery_embeddings)


def _gather_sc(imgs_h, masks_h, idx_flat):
    nrow = idx_flat.shape[0] // 8
    si = imgs_h.shape[1]
    sm = masks_h.shape[1]
    mesh = plsc.VectorSubcoreMesh(core_axis_name="c", subcore_axis_name="s")

    @functools.partial(
        pl.kernel,
        out_type=(
            jax.ShapeDtypeStruct((nrow, 2, si, 128), jnp.float32),
            jax.ShapeDtypeStruct((nrow, 2, sm, 128), jnp.float32),
        ),
        mesh=mesh,
        compiler_params=pltpu.CompilerParams(use_tc_tiling_on_sc=True),
        scratch_types=[
            pltpu.VMEM((8,), jnp.int32),
            pltpu.VMEM((2, si, 128), jnp.float32),
            pltpu.VMEM((2, sm, 128), jnp.float32),
            pltpu.SemaphoreType.DMA,
            pltpu.SemaphoreType.DMA,
        ],
    )
    def k(imgs_hbm, masks_hbm, idx_hbm, oi_hbm, om_hbm,
          idx_v, img_rows, msk_rows, sem1, sem2):
        wid = lax.axis_index("s") * 2 + lax.axis_index("c")
        pltpu.sync_copy(idx_hbm.at[pl.ds(wid * 8, 8)], idx_v)
        cp1 = pltpu.async_copy(imgs_hbm.at[idx_v.at[pl.ds(0, 2)]], img_rows, sem1)
        cp2 = pltpu.async_copy(masks_hbm.at[idx_v.at[pl.ds(0, 2)]], msk_rows, sem2)
        cp1.wait()
        pltpu.sync_copy(img_rows, oi_hbm.at[wid])
        cp2.wait()
        pltpu.sync_copy(msk_rows, om_hbm.at[wid])

    return k(imgs_h, masks_h, idx_flat)


def _combine_body(w_ref, xi_ref, xm_ref, oi_ref, om_ref):
    w0 = w_ref[:, 0]
    w1 = w_ref[:, 1]
    oi_ref[...] = xi_ref[:, 0] * w0 + xi_ref[:, 1] * w1
    om_ref[...] = xm_ref[:, 0] * w0 + xm_ref[:, 1] * w1


def _combine_tc(w_splat, rows_img, rows_msk):
    nrow, _, si, _ = rows_img.shape
    sm = rows_msk.shape[2]
    return pl.pallas_call(
        _combine_body,
        out_shape=(
            jax.ShapeDtypeStruct((nrow, si, 128), jnp.float32),
            jax.ShapeDtypeStruct((nrow, sm, 128), jnp.float32),
        ),
    )(w_splat, rows_img, rows_msk)


def kernel(query_encoding, W1, b1, W2, b2, gallery_embeddings,
           gallery_images, gallery_masks):
    B = query_encoding.shape[0]
    N, C, H, W = gallery_images.shape
    D = C * H * W
    M = H * W
    si = D // (2 * 128)
    sm = M // (2 * 128)
    R = 2 * B

    topk_idx, topk_w = _topk_tc(query_encoding, W1, b1, W2, b2,
                                gallery_embeddings)

    rep_idx = jnp.repeat(topk_idx, 2, axis=0)
    half = (jnp.arange(R, dtype=jnp.int32) % 2)[:, None]
    idx_flat = jnp.pad(2 * rep_idx + half, ((0, 0), (0, 6))).reshape(-1)
    w_splat = jnp.broadcast_to(
        jnp.repeat(topk_w, 2, axis=0)[:, :, None, None], (R, 2, 1, 128))

    imgs_h = gallery_images.reshape(2 * N, si, 128)
    masks_h = gallery_masks.reshape(2 * N, sm, 128)
    rows_img, rows_msk = _gather_sc(imgs_h, masks_h, idx_flat)

    gi, gm = _combine_tc(w_splat, rows_img, rows_msk)

    guide_image = gi.reshape(B, C, H, W)
    guide_mask = gm.reshape(B, H, W)
    return guide_image, guide_mask


# speedup vs baseline: 11.2386x; 11.0319x over previous
import functools

import jax
import jax.numpy as jnp
from jax import lax
from jax.experimental import pallas as pl
from jax.experimental.pallas import tpu as pltpu
from jax.experimental.pallas import tpu_sc as plsc

_TAU = 0.1
_EPS = 1e-8
_L = 16
_TR = 8
_TC = 128


def _topk_body(x_ref, w1_ref, b1_ref, w2_ref, b2_ref, g_ref, idx_ref, w_ref):
    x = x_ref[...]
    h = jnp.maximum(
        lax.dot_general(x, w1_ref[...], (((1,), (0,)), ((), ())),
                        preferred_element_type=jnp.float32)
        + b1_ref[...][None, :], 0.0)
    q = lax.dot_general(h, w2_ref[...], (((1,), (0,)), ((), ())),
                        preferred_element_type=jnp.float32) + b2_ref[...][None, :]
    qn = q / jnp.maximum(jnp.sqrt(jnp.sum(q * q, axis=1, keepdims=True)), _EPS)
    g = g_ref[...]
    gn = g / jnp.maximum(jnp.sqrt(jnp.sum(g * g, axis=1, keepdims=True)), _EPS)
    sim = lax.dot_general(qn, gn, (((1,), (1,)), ((), ())),
                          preferred_element_type=jnp.float32)
    n = sim.shape[1]
    ii = lax.broadcasted_iota(jnp.int32, sim.shape, 1)
    m1 = jnp.max(sim, axis=1, keepdims=True)
    i1 = jnp.min(jnp.where(sim == m1, ii, n), axis=1, keepdims=True)
    sim2 = jnp.where(ii == i1, -jnp.inf, sim)
    m2 = jnp.max(sim2, axis=1, keepdims=True)
    i2 = jnp.min(jnp.where(sim2 == m2, ii, n), axis=1, keepdims=True)
    e = jnp.exp((m2 - m1) / _TAU)
    denom = 1.0 + e
    idx_ref[...] = jnp.concatenate([i1, i2], axis=1)
    w_ref[...] = jnp.concatenate([1.0 / denom, e / denom], axis=1)


def _topk_tc(query_encoding, W1, b1, W2, b2, gallery_embeddings):
    B = query_encoding.shape[0]
    return pl.pallas_call(
        _topk_body,
        out_shape=(
            jax.ShapeDtypeStruct((B, 2), jnp.int32),
            jax.ShapeDtypeStruct((B, 2), jnp.float32),
        ),
    )(query_encoding, W1, b1, W2, b2, gallery_embeddings)


def _gather_combine_sc(imgs_flat, masks_flat, ai_flat, am_flat, wb, dh, mh):
    nrow = wb.shape[0]
    mesh = plsc.VectorSubcoreMesh(core_axis_name="c", subcore_axis_name="s")

    @functools.partial(
        pl.kernel,
        out_type=(
            jax.ShapeDtypeStruct((nrow, dh), jnp.float32),
            jax.ShapeDtypeStruct((nrow, mh), jnp.float32),
        ),
        mesh=mesh,
        scratch_types=[
            pltpu.VMEM((2, _L), jnp.float32),
            pltpu.VMEM((dh,), jnp.int32),
            pltpu.VMEM((dh,), jnp.int32),
            pltpu.VMEM((dh,), jnp.float32),
            pltpu.VMEM((dh,), jnp.float32),
            pltpu.VMEM((mh,), jnp.int32),
            pltpu.VMEM((mh,), jnp.int32),
            pltpu.VMEM((mh,), jnp.float32),
            pltpu.VMEM((mh,), jnp.float32),
            pltpu.VMEM((dh,), jnp.float32),
            pltpu.VMEM((mh,), jnp.float32),
            pltpu.SemaphoreType.DMA,
            pltpu.SemaphoreType.DMA,
        ],
    )
    def k(imgs_hbm, masks_hbm, ai_hbm, am_hbm, wb_hbm, oi_hbm, om_hbm,
          w_v, ii0, ii1, ci0, ci1, im0, im1, cm0, cm1,
          acc_i, acc_m, sem1, sem2):
        wid = lax.axis_index("s") * 2 + lax.axis_index("c")
        pltpu.sync_copy(wb_hbm.at[wid], w_v)
        cps = []
        for kk, (ib, cb, mb, nb) in enumerate(
                ((ii0, ci0, im0, cm0), (ii1, ci1, im1, cm1))):
            pltpu.sync_copy(ai_hbm.at[pl.ds((wid * 2 + kk) * dh, dh)], ib)
            cps.append(pltpu.async_copy(imgs_hbm.at[ib], cb, sem1))
            pltpu.sync_copy(am_hbm.at[pl.ds((wid * 2 + kk) * mh, mh)], mb)
            cps.append(pltpu.async_copy(masks_hbm.at[mb], nb, sem2))

        w0 = w_v[0]
        w1 = w_v[1]
        cps[0].wait()
        cps[2].wait()

        def ci(j, carry):
            s = pl.ds(j * _L, _L)
            acc_i[s] = w0 * ci0[s] + w1 * ci1[s]
            return carry

        lax.fori_loop(0, dh // _L, ci, 0)
        pltpu.sync_copy(acc_i, oi_hbm.at[wid])
        cps[1].wait()
        cps[3].wait()

        def cm(j, carry):
            s = pl.ds(j * _L, _L)
            acc_m[s] = w0 * cm0[s] + w1 * cm1[s]
            return carry

        lax.fori_loop(0, mh // _L, cm, 0)
        pltpu.sync_copy(acc_m, om_hbm.at[wid])

    return k(imgs_flat, masks_flat, ai_flat, am_flat, wb)


def kernel(query_encoding, W1, b1, W2, b2, gallery_embeddings,
           gallery_images, gallery_masks):
    B = query_encoding.shape[0]
    N, C, H, W = gallery_images.shape
    D = C * H * W
    M = H * W
    R = 2 * B

    topk_idx, topk_w = _topk_tc(query_encoding, W1, b1, W2, b2,
                                gallery_embeddings)

    imgs_flat = (gallery_images.transpose(1, 2, 3, 0)
                 .reshape(C, H, W // _TR, _TR, N // _TC, _TC)
                 .transpose(0, 1, 2, 4, 3, 5).reshape(-1))
    masks_flat = (gallery_masks.transpose(1, 2, 0)
                  .reshape(H, W // _TR, _TR, N // _TC, _TC)
                  .transpose(0, 1, 3, 2, 4).reshape(-1))
    dh, mh = D // 2, M // 2
    rep_idx = jnp.repeat(topk_idx, 2, axis=0)
    wb = jnp.broadcast_to(
        jnp.repeat(topk_w, 2, axis=0)[:, :, None], (R, 2, _L))
    half = (jnp.arange(R, dtype=jnp.int32) % 2)
    key = (rep_idx // _TC) * 1024 + rep_idx % _TC
    qi = half[:, None, None] * dh + jnp.arange(dh, dtype=jnp.int32)
    ai = (qi // _TR) * 32768 + (qi % _TR) * _TC + key[:, :, None]
    qm = half[:, None, None] * mh + jnp.arange(mh, dtype=jnp.int32)
    am = (qm // _TR) * 32768 + (qm % _TR) * _TC + key[:, :, None]

    gi, gm = _gather_combine_sc(imgs_flat, masks_flat,
                                ai.reshape(-1), am.reshape(-1), wb, dh, mh)

    guide_image = gi.reshape(B, C, H, W)
    guide_mask = gm.reshape(B, H, W)
    return guide_image, guide_mask
